# baseline (device time: 8201 ns/iter reference)
import jax
import jax.numpy as jnp
from jax import lax
from jax.experimental import pallas as pl
from jax.experimental.pallas import tpu as pltpu


def kernel(x, dy, gamma):
    _, d = x.shape

    def body(x_hbm, dy_hbm, gamma_hbm, out_hbm,
             x_v, dy_v, acc_v, comm_ref, copy_sems, send_sem, recv_sem):
        my_x = lax.axis_index("x")
        my_y = lax.axis_index("y")
        peer = (my_x, 1 - my_y)

        barrier_sem = pltpu.get_barrier_semaphore()
        pl.semaphore_signal(
            barrier_sem, inc=1, device_id=peer,
            device_id_type=pl.DeviceIdType.MESH,
        )

        cp_x = pltpu.make_async_copy(x_hbm, x_v, copy_sems.at[0])
        cp_dy = pltpu.make_async_copy(dy_hbm, dy_v, copy_sems.at[1])
        cp_x.start()
        cp_dy.start()

        cp_x.wait()
        xv = x_v[:, :]
        inv_d = 1.0 / d
        mu = jnp.sum(xv, axis=1, keepdims=True) * inv_d
        msq = jnp.sum(xv * xv, axis=1, keepdims=True) * inv_d
        rstd = lax.rsqrt(msq - mu * mu + 1e-5)

        cp_dy.wait()
        dyv = dy_v[:, :]
        comm_ref[0, 0, :] = jnp.sum((rstd * xv - mu * rstd) * dyv, axis=0)
        comm_ref[0, 1, :] = jnp.sum(dyv, axis=0)

        pl.semaphore_wait(barrier_sem, 1)

        rdma = pltpu.make_async_remote_copy(
            src_ref=comm_ref.at[0],
            dst_ref=comm_ref.at[1],
            send_sem=send_sem,
            recv_sem=recv_sem,
            device_id=peer,
            device_id_type=pl.DeviceIdType.MESH,
        )
        rdma.start()
        rdma.wait()

        acc_v[:, :] = comm_ref[0] + comm_ref[1]
        cp_out = pltpu.make_async_copy(acc_v, out_hbm, copy_sems.at[2])
        cp_out.start()
        cp_out.wait()

    return pl.pallas_call(
        body,
        out_shape=jax.ShapeDtypeStruct((2, d), jnp.float32),
        in_specs=[pl.BlockSpec(memory_space=pl.ANY)] * 3,
        out_specs=pl.BlockSpec(memory_space=pl.ANY),
        scratch_shapes=[
            pltpu.VMEM((512, d), jnp.float32),
            pltpu.VMEM((512, d), jnp.float32),
            pltpu.VMEM((2, d), jnp.float32),
            pltpu.VMEM((2, 2, d), jnp.float32),
            pltpu.SemaphoreType.DMA((3,)),
            pltpu.SemaphoreType.DMA,
            pltpu.SemaphoreType.DMA,
        ],
        compiler_params=pltpu.CompilerParams(collective_id=0),
    )(x, dy, gamma)


# device time: 6164 ns/iter; 1.3305x vs baseline; 1.3305x over previous
import jax
import jax.numpy as jnp
from jax import lax
from jax.experimental import pallas as pl
from jax.experimental.pallas import tpu as pltpu


def kernel(x, dy, gamma):
    _, d = x.shape

    def body(x_hbm, dy_hbm, gamma_hbm, out_hbm,
             x_v, dy_v, acc_v, comm_ref, copy_sems, send_sem, recv_sem):
        my_x = lax.axis_index("x")
        my_y = lax.axis_index("y")
        peer = (my_x, 1 - my_y)

        barrier_sem = pltpu.get_barrier_semaphore()
        pl.semaphore_signal(
            barrier_sem, inc=1, device_id=peer,
            device_id_type=pl.DeviceIdType.MESH,
        )

        cp_x = pltpu.make_async_copy(x_hbm, x_v, copy_sems.at[0])
        cp_dy = pltpu.make_async_copy(dy_hbm, dy_v, copy_sems.at[1])
        cp_x.start()
        cp_dy.start()

        cp_x.wait()
        xv = x_v[:, :]
        inv_d = 1.0 / d
        mu = jnp.sum(xv, axis=1, keepdims=True) * inv_d
        msq = jnp.sum(xv * xv, axis=1, keepdims=True) * inv_d
        rstd = lax.rsqrt(msq - mu * mu + 1e-5)

        cp_dy.wait()
        dyv = dy_v[:, :]
        comm_ref[0, 0, :] = jnp.sum((rstd * xv - mu * rstd) * dyv, axis=0)
        comm_ref[0, 1, :] = jnp.sum(dyv, axis=0)

        pl.semaphore_wait(barrier_sem, 1)

        rdma = pltpu.make_async_remote_copy(
            src_ref=comm_ref.at[0],
            dst_ref=comm_ref.at[1],
            send_sem=send_sem,
            recv_sem=recv_sem,
            device_id=peer,
            device_id_type=pl.DeviceIdType.MESH,
        )
        rdma.start()
        rdma.wait()

        acc_v[:, :] = comm_ref[0] + comm_ref[1]
        cp_out = pltpu.make_async_copy(acc_v, out_hbm, copy_sems.at[2])
        cp_out.start()
        cp_out.wait()

    return pl.pallas_call(
        body,
        out_shape=jax.ShapeDtypeStruct((2, d), jnp.float32),
        in_specs=[pl.BlockSpec(memory_space=pltpu.MemorySpace.HBM)] * 3,
        out_specs=pl.BlockSpec(memory_space=pltpu.MemorySpace.HBM),
        scratch_shapes=[
            pltpu.VMEM((512, d), jnp.float32),
            pltpu.VMEM((512, d), jnp.float32),
            pltpu.VMEM((2, d), jnp.float32),
            pltpu.VMEM((2, 2, d), jnp.float32),
            pltpu.SemaphoreType.DMA((3,)),
            pltpu.SemaphoreType.DMA,
            pltpu.SemaphoreType.DMA,
        ],
        compiler_params=pltpu.CompilerParams(collective_id=0),
    )(
        pltpu.with_memory_space_constraint(x, pltpu.MemorySpace.HBM),
        pltpu.with_memory_space_constraint(dy, pltpu.MemorySpace.HBM),
        pltpu.with_memory_space_constraint(gamma, pltpu.MemorySpace.HBM),
    )


# device time: 6159 ns/iter; 1.3315x vs baseline; 1.0008x over previous
import jax
import jax.numpy as jnp
from jax import lax
from jax.experimental import pallas as pl
from jax.experimental.pallas import tpu as pltpu


def kernel(x, dy, gamma):
    _, d = x.shape

    def body(x_hbm, dy_hbm, gamma_hbm, out_ref,
             x_v, dy_v, comm_ref, copy_sems, send_sem, recv_sem):
        my_x = lax.axis_index("x")
        my_y = lax.axis_index("y")
        peer = (my_x, 1 - my_y)

        barrier_sem = pltpu.get_barrier_semaphore()
        pl.semaphore_signal(
            barrier_sem, inc=1, device_id=peer,
            device_id_type=pl.DeviceIdType.MESH,
        )

        cp_x = pltpu.make_async_copy(x_hbm, x_v, copy_sems.at[0])
        cp_dy = pltpu.make_async_copy(dy_hbm, dy_v, copy_sems.at[1])
        cp_x.start()
        cp_dy.start()

        cp_x.wait()
        xv = x_v[:, :]
        inv_d = 1.0 / d
        mu = jnp.sum(xv, axis=1, keepdims=True) * inv_d
        msq = jnp.sum(xv * xv, axis=1, keepdims=True) * inv_d
        rstd = lax.rsqrt(msq - mu * mu + 1e-5)

        cp_dy.wait()
        dyv = dy_v[:, :]
        comm_ref[0, 0, :] = jnp.sum((rstd * xv - mu * rstd) * dyv, axis=0)
        comm_ref[0, 1, :] = jnp.sum(dyv, axis=0)

        pl.semaphore_wait(barrier_sem, 1)

        rdma = pltpu.make_async_remote_copy(
            src_ref=comm_ref.at[0],
            dst_ref=comm_ref.at[1],
            send_sem=send_sem,
            recv_sem=recv_sem,
            device_id=peer,
            device_id_type=pl.DeviceIdType.MESH,
        )
        rdma.start()
        rdma.wait()

        out_ref[:, :] = comm_ref[0] + comm_ref[1]

    return pl.pallas_call(
        body,
        out_shape=jax.ShapeDtypeStruct((2, d), jnp.float32),
        in_specs=[pl.BlockSpec(memory_space=pltpu.MemorySpace.HBM)] * 3,
        out_specs=pl.BlockSpec(memory_space=pltpu.VMEM),
        scratch_shapes=[
            pltpu.VMEM((512, d), jnp.float32),
            pltpu.VMEM((512, d), jnp.float32),
            pltpu.VMEM((2, 2, d), jnp.float32),
            pltpu.SemaphoreType.DMA((2,)),
            pltpu.SemaphoreType.DMA,
            pltpu.SemaphoreType.DMA,
        ],
        compiler_params=pltpu.CompilerParams(collective_id=0),
    )(
        pltpu.with_memory_space_constraint(x, pltpu.MemorySpace.HBM),
        pltpu.with_memory_space_constraint(dy, pltpu.MemorySpace.HBM),
        pltpu.with_memory_space_constraint(gamma, pltpu.MemorySpace.HBM),
    )


# device time: 6136 ns/iter; 1.3365x vs baseline; 1.0037x over previous
import jax
import jax.numpy as jnp
from jax import lax
from jax.experimental import pallas as pl
from jax.experimental.pallas import tpu as pltpu


def kernel(x, dy, gamma):
    _, d = x.shape

    def body(x_hbm, dy_hbm, gamma_hbm, out_ref,
             x_v, dy_v, comm_ref, copy_sems, send_sem, recv_sem):
        my_x = lax.axis_index("x")
        my_y = lax.axis_index("y")
        peer = (my_x, 1 - my_y)

        barrier_sem = pltpu.get_barrier_semaphore()
        pl.semaphore_signal(
            barrier_sem, inc=1, device_id=peer,
            device_id_type=pl.DeviceIdType.MESH,
        )

        n_chunks = 2
        rows = x_v.shape[0] // n_chunks
        cps = []
        for c in range(n_chunks):
            sl = pl.ds(c * rows, rows)
            cp_x = pltpu.make_async_copy(
                x_hbm.at[sl], x_v.at[sl], copy_sems.at[2 * c])
            cp_dy = pltpu.make_async_copy(
                dy_hbm.at[sl], dy_v.at[sl], copy_sems.at[2 * c + 1])
            cp_x.start()
            cp_dy.start()
            cps.append((cp_x, cp_dy))

        inv_d = 1.0 / d
        dg = None
        db = None
        for c in range(n_chunks):
            cp_x, cp_dy = cps[c]
            sl = pl.ds(c * rows, rows)
            cp_x.wait()
            xv = x_v[sl, :]
            mu = jnp.sum(xv, axis=1, keepdims=True) * inv_d
            msq = jnp.sum(xv * xv, axis=1, keepdims=True) * inv_d
            rstd = lax.rsqrt(msq - mu * mu + 1e-5)
            cp_dy.wait()
            dyv = dy_v[sl, :]
            dg_c = jnp.sum((rstd * xv - mu * rstd) * dyv, axis=0)
            db_c = jnp.sum(dyv, axis=0)
            dg = dg_c if dg is None else dg + dg_c
            db = db_c if db is None else db + db_c
        comm_ref[0, 0, :] = dg
        comm_ref[0, 1, :] = db

        pl.semaphore_wait(barrier_sem, 1)

        rdma = pltpu.make_async_remote_copy(
            src_ref=comm_ref.at[0],
            dst_ref=comm_ref.at[1],
            send_sem=send_sem,
            recv_sem=recv_sem,
            device_id=peer,
            device_id_type=pl.DeviceIdType.MESH,
        )
        rdma.start()
        rdma.wait()

        out_ref[:, :] = comm_ref[0] + comm_ref[1]

    return pl.pallas_call(
        body,
        out_shape=jax.ShapeDtypeStruct((2, d), jnp.float32),
        in_specs=[pl.BlockSpec(memory_space=pltpu.MemorySpace.HBM)] * 3,
        out_specs=pl.BlockSpec(memory_space=pltpu.VMEM),
        scratch_shapes=[
            pltpu.VMEM((512, d), jnp.float32),
            pltpu.VMEM((512, d), jnp.float32),
            pltpu.VMEM((2, 2, d), jnp.float32),
            pltpu.SemaphoreType.DMA((4,)),
            pltpu.SemaphoreType.DMA,
            pltpu.SemaphoreType.DMA,
        ],
        compiler_params=pltpu.CompilerParams(collective_id=0),
    )(
        pltpu.with_memory_space_constraint(x, pltpu.MemorySpace.HBM),
        pltpu.with_memory_space_constraint(dy, pltpu.MemorySpace.HBM),
        pltpu.with_memory_space_constraint(gamma, pltpu.MemorySpace.HBM),
    )


# device time: 6128 ns/iter; 1.3383x vs baseline; 1.0013x over previous
import jax
import jax.numpy as jnp
from jax import lax
from jax.experimental import pallas as pl
from jax.experimental.pallas import tpu as pltpu


def kernel(x, dy, gamma):
    _, d = x.shape

    def body(x_hbm, dy_hbm, gamma_hbm, out_ref,
             x_v, dy_v, comm_ref, copy_sems, send_sem, recv_sem):
        my_x = lax.axis_index("x")
        my_y = lax.axis_index("y")
        peer = (my_x, 1 - my_y)

        barrier_sem = pltpu.get_barrier_semaphore()
        pl.semaphore_signal(
            barrier_sem, inc=1, device_id=peer,
            device_id_type=pl.DeviceIdType.MESH,
        )

        n_chunks = 2
        rows = x_v.shape[0] // n_chunks
        cps = []
        for c in range(n_chunks):
            sl = pl.ds(c * rows, rows)
            cp_x = pltpu.make_async_copy(
                x_hbm.at[sl], x_v.at[sl], copy_sems.at[2 * c])
            cp_dy = pltpu.make_async_copy(
                dy_hbm.at[sl], dy_v.at[sl], copy_sems.at[2 * c + 1])
            cp_x.start()
            cp_dy.start()
            cps.append((cp_x, cp_dy))

        inv_d = 1.0 / d
        dg = None
        db = None
        for c in range(n_chunks):
            cp_x, cp_dy = cps[c]
            sl = pl.ds(c * rows, rows)
            cp_x.wait()
            xv = x_v[sl, :]
            mu = jnp.sum(xv, axis=1, keepdims=True) * inv_d
            msq = jnp.sum(xv * xv, axis=1, keepdims=True) * inv_d
            rstd = lax.rsqrt(msq - mu * mu + 1e-5)
            cp_dy.wait()
            dyv = dy_v[sl, :]
            ones = jnp.ones((1, rows), jnp.float32)
            wdy = (rstd * xv - mu * rstd) * dyv
            dg_c = jnp.dot(ones, wdy, preferred_element_type=jnp.float32)[0]
            db_c = jnp.dot(ones, dyv, preferred_element_type=jnp.float32)[0]
            dg = dg_c if dg is None else dg + dg_c
            db = db_c if db is None else db + db_c
        comm_ref[0, 0, :] = dg
        comm_ref[0, 1, :] = db

        pl.semaphore_wait(barrier_sem, 1)

        rdma = pltpu.make_async_remote_copy(
            src_ref=comm_ref.at[0],
            dst_ref=comm_ref.at[1],
            send_sem=send_sem,
            recv_sem=recv_sem,
            device_id=peer,
            device_id_type=pl.DeviceIdType.MESH,
        )
        rdma.start()
        rdma.wait()

        out_ref[:, :] = comm_ref[0] + comm_ref[1]

    return pl.pallas_call(
        body,
        out_shape=jax.ShapeDtypeStruct((2, d), jnp.float32),
        in_specs=[pl.BlockSpec(memory_space=pltpu.MemorySpace.HBM)] * 3,
        out_specs=pl.BlockSpec(memory_space=pltpu.VMEM),
        scratch_shapes=[
            pltpu.VMEM((512, d), jnp.float32),
            pltpu.VMEM((512, d), jnp.float32),
            pltpu.VMEM((2, 2, d), jnp.float32),
            pltpu.SemaphoreType.DMA((4,)),
            pltpu.SemaphoreType.DMA,
            pltpu.SemaphoreType.DMA,
        ],
        compiler_params=pltpu.CompilerParams(collective_id=0),
    )(
        pltpu.with_memory_space_constraint(x, pltpu.MemorySpace.HBM),
        pltpu.with_memory_space_constraint(dy, pltpu.MemorySpace.HBM),
        pltpu.with_memory_space_constraint(gamma, pltpu.MemorySpace.HBM),
    )
